# baseline (speedup 1.0000x reference)
import functools

import jax
import jax.numpy as jnp
from jax import lax
from jax.experimental import pallas as pl
from jax.experimental.pallas import tpu as pltpu
from jax.experimental.pallas import tpu_sc as plsc

B, S, H = 4, 8192, 768
E = 64
K = 8
N = B * S
C = 4
NTC = N // C
NW = 32
TPW = NTC // NW
BT = 1024
NBC = NTC // BT
AUX_COEF = 0.01
Z_COEF = 0.01
L = 16


def _tc_body(x_ref, wt_ref, pt_ref, stats_ref):
    b = pl.program_id(0)
    x = x_ref[...]
    lg = jnp.dot(x, wt_ref[...], preferred_element_type=jnp.float32)
    lg = jnp.clip(lg, -10.0, 10.0)
    m = jnp.max(lg, axis=1, keepdims=True)
    ex = jnp.exp(lg - m)
    s = jnp.sum(ex, axis=1, keepdims=True)
    probs = ex / s
    pt_ref[...] = probs.T
    logz = m + jnp.log(s)

    @pl.when(b == 0)
    def _():
        stats_ref[...] = jnp.zeros_like(stats_ref)

    stats_ref[0:1, 0:E] += jnp.sum(probs, axis=0, keepdims=True)
    stats_ref[1:2, 0:1] += jnp.sum(logz * logz, axis=0, keepdims=True)


def _make_tc_call(c):
    return pl.pallas_call(
        _tc_body,
        grid=(NBC,),
        in_specs=[
            pl.BlockSpec((BT, H), lambda b: (c * NBC + b, 0)),
            pl.BlockSpec((H, E), lambda b: (0, 0)),
        ],
        out_specs=[
            pl.BlockSpec((E, BT), lambda b: (0, b)),
            pl.BlockSpec((8, 128), lambda b: (0, 0)),
        ],
        out_shape=[
            jax.ShapeDtypeStruct((E, NTC), jnp.float32),
            jax.ShapeDtypeStruct((8, 128), jnp.float32),
        ],
    )


_tc_calls = [_make_tc_call(c) for c in range(C)]


_NET8 = [(0, 1), (2, 3), (4, 5), (6, 7), (0, 2), (1, 3), (4, 6), (5, 7),
         (1, 2), (5, 6), (0, 4), (3, 7), (1, 5), (2, 6), (1, 4), (3, 6),
         (2, 4), (3, 5), (3, 4)]


def _ce(a, b):
    av, ai = a
    bv, bi = b
    p = av > bv
    return (
        (jnp.maximum(av, bv), jnp.where(p, ai, bi)),
        (jnp.minimum(av, bv), jnp.where(p, bi, ai)),
    )


def _sort8(g):
    g = list(g)
    for i, j in _NET8:
        g[i], g[j] = _ce(g[i], g[j])
    return g


def _merge8(a, b):
    c = [_ce(a[i], b[7 - i])[0] for i in range(8)]
    for step in (4, 2, 1):
        nc = list(c)
        for i in range(8):
            j = i ^ step
            if i < j:
                nc[i], nc[j] = _ce(c[i], c[j])
        c = nc
    return c


_sc_mesh = plsc.VectorSubcoreMesh(
    core_axis_name="c", subcore_axis_name="s", num_cores=2, num_subcores=16
)


@functools.partial(
    pl.kernel,
    out_type=[
        jax.ShapeDtypeStruct((K, NTC), jnp.float32),
        jax.ShapeDtypeStruct((K, NTC), jnp.int32),
        jax.ShapeDtypeStruct((NW, E, L), jnp.float32),
    ],
    mesh=_sc_mesh,
    scratch_types=[
        pltpu.VMEM((E, TPW), jnp.float32),
        pltpu.VMEM((K, TPW), jnp.float32),
        pltpu.VMEM((K, TPW), jnp.int32),
        pltpu.VMEM((E, L), jnp.float32),
    ],
)
def _sc_topk(pt_hbm, w_hbm, i_hbm, c_hbm, in_v, w_v, i_v, cacc_v):
    wid = lax.axis_index("s") * 2 + lax.axis_index("c")
    col0 = wid * TPW
    pltpu.sync_copy(pt_hbm.at[:, pl.ds(col0, TPW)], in_v)

    zero16 = jnp.zeros((L,), jnp.float32)
    for e in range(E):
        cacc_v[e] = zero16

    def group(g, _):
        t0 = g * L

        def sorted_group(g8):
            e0 = g8 * 8
            return _sort8([
                (in_v[e0 + i, pl.ds(t0, L)],
                 jnp.full((L,), float(e0 + i), jnp.float32))
                for i in range(8)
            ])

        m01 = _merge8(sorted_group(0), sorted_group(1))
        m23 = _merge8(sorted_group(2), sorted_group(3))
        m03 = _merge8(m01, m23)
        m45 = _merge8(sorted_group(4), sorted_group(5))
        m67 = _merge8(sorted_group(6), sorted_group(7))
        m47 = _merge8(m45, m67)
        top = _merge8(m03, m47)

        ssum = top[0][0]
        for j in range(1, K):
            ssum = ssum + top[j][0]
        inv = 1.0 / ssum
        for j in range(K):
            w_v[j, pl.ds(t0, L)] = top[j][0] * inv
            i_v[j, pl.ds(t0, L)] = top[j][1].astype(jnp.int32)

        tvv = top[K - 1][0]
        tvi = top[K - 1][1]
        one16 = jnp.ones((L,), jnp.float32)
        for e in range(E):
            x = in_v[e, pl.ds(t0, L)]
            sel = (x > tvv) | ((x == tvv) & (float(e) <= tvi))
            cacc_v[e] += jnp.where(sel, one16, zero16)
        return 0

    lax.fori_loop(0, TPW // L, group, 0)

    pltpu.sync_copy(w_v, w_hbm.at[:, pl.ds(col0, TPW)])
    pltpu.sync_copy(i_v, i_hbm.at[:, pl.ds(col0, TPW)])
    pltpu.sync_copy(cacc_v, c_hbm.at[wid])


def kernel(hidden_states, W):
    x = hidden_states.reshape(N, H)
    wt = W.T
    pts, stats, outs = [], [], []
    pts.append(_tc_calls[0](x, wt))
    for c in range(1, C):
        pts.append(_tc_calls[c](x, wt))
        outs.append(_sc_topk(pts[c - 1][0]))
    outs.append(_sc_topk(pts[C - 1][0]))

    ws = jnp.stack([o[0] for o in outs])
    ids = jnp.stack([o[1] for o in outs])
    router_weights = ws.transpose(0, 2, 1).reshape(B, S, K)
    selected_experts = ids.transpose(0, 2, 1).reshape(B, S, K)
    cnts = sum(o[2].sum(axis=(0, 2)) for o in outs)
    st = sum(p[1] for p in pts)
    tokens_per_expert = cnts / N
    router_prob_per_expert = st[0, :E] / N
    load_balancing_loss = (
        E * jnp.sum(tokens_per_expert * router_prob_per_expert) * AUX_COEF
    )
    router_z_loss = st[1, 0] / N * Z_COEF
    return router_weights, selected_experts, load_balancing_loss, router_z_loss
